# Initial kernel scaffold; baseline (speedup 1.0000x reference)
import jax
import jax.numpy as jnp
from jax.experimental import pallas as pl


def kernel(x, edge_index, batch, c1_w1, c1_b1, c1_w2, c1_b2, c2_w1, c2_b1, c2_w2, c2_b2, c3_w1, c3_b1, c3_w2, c3_b2, l1_w, l1_b, l2_w, l2_b):
    raise NotImplementedError("write your pallas kernel here")


# speedup vs baseline: 4.5502x; 4.5502x over previous
import functools

import jax
import jax.numpy as jnp
from jax import lax
from jax.experimental import pallas as pl
from jax.experimental.pallas import tpu as pltpu
from jax.experimental.pallas import tpu_sc as plsc

_N = 10000
_E = 160000
_HID = 256
_H2 = 128
_G = 64
_OUT = 3
_NC = 2
_NS = 16
_EPW = _E // _NS
_CH = 128
_NCHUNK = -(-_EPW // _CH)
_STRIPE = 624
_STRIPE_LAST = _N - 15 * _STRIPE
_NPAD = _N + 8

def _sc_agg_body(table_hbm, sidx_hbm, didx_hbm, out_hbm, slab, sidx_v, didx_v,
                 rows_v, sem):
    c = lax.axis_index("c")
    s = lax.axis_index("s")
    base = s * _STRIPE
    @pl.when(s < _NS - 1)
    def _():
        pltpu.sync_copy(
            table_hbm.at[pl.ds(c * _N + base, _STRIPE)],
            slab.at[pl.ds(base, _STRIPE)],
        )

    @pl.when(s == _NS - 1)
    def _():
        pltpu.sync_copy(
            table_hbm.at[pl.ds(c * _N + base, _STRIPE_LAST)],
            slab.at[pl.ds(base, _STRIPE_LAST)],
        )

    pltpu.sync_copy(sidx_hbm.at[c, s], sidx_v)
    pltpu.sync_copy(didx_hbm.at[s], didx_v)
    plsc.subcore_barrier()

    def step(j, carry):
        pltpu.async_copy(table_hbm.at[sidx_v.at[j]], rows_v, sem).wait()
        pltpu.sync_copy(rows_v, slab.at[didx_v.at[j]], add=True)
        return carry

    lax.fori_loop(0, _NCHUNK, step, 0)
    plsc.subcore_barrier()

    @pl.when(s < _NS - 1)
    def _():
        pltpu.sync_copy(
            slab.at[pl.ds(base, _STRIPE)],
            out_hbm.at[pl.ds(c * _N + base, _STRIPE)],
        )

    @pl.when(s == _NS - 1)
    def _():
        pltpu.sync_copy(
            slab.at[pl.ds(base, _STRIPE_LAST)],
            out_hbm.at[pl.ds(c * _N + base, _STRIPE_LAST)],
        )


@functools.lru_cache(maxsize=1)
def _sc_agg_kernel():
    mesh = plsc.VectorSubcoreMesh(
        core_axis_name="c", subcore_axis_name="s",
        num_cores=_NC, num_subcores=_NS,
    )
    return pl.kernel(
        _sc_agg_body,
        out_type=jax.ShapeDtypeStruct((2 * _N, _H2), jnp.float32),
        mesh=mesh,
        scratch_types=[
            pltpu.VMEM_SHARED((_NPAD, _H2), jnp.float32),
            pltpu.VMEM((_NCHUNK, _CH), jnp.int32),
            pltpu.VMEM((_NCHUNK, _CH), jnp.int32),
            pltpu.VMEM((_CH, _H2), jnp.float32),
            pltpu.SemaphoreType.DMA,
        ],
    )


def _sc_agg(table, sidx, didx):
    return _sc_agg_kernel()(table, sidx, didx)


_BR = 1000


def _mlp_body(z_ref, w1_ref, b1_ref, w2_ref, b2_ref, out_ref):
    z0 = z_ref[0]
    z1 = z_ref[1]
    t = jnp.dot(z0, w1_ref[:_H2, :], preferred_element_type=jnp.float32)
    t = t + jnp.dot(z1, w1_ref[_H2:, :], preferred_element_type=jnp.float32)
    t = jnp.maximum(t + b1_ref[0][None, :], 0.0)
    y = jnp.dot(t, w2_ref[...], preferred_element_type=jnp.float32)
    y = jnp.maximum(y + b2_ref[0][None, :], 0.0)
    out_ref[0] = y[:, :_H2]
    out_ref[1] = y[:, _H2:]


def _mlp(z, w1, b1, w2, b2):
    return pl.pallas_call(
        _mlp_body,
        grid=(_N // _BR,),
        in_specs=[
            pl.BlockSpec((2, _BR, _H2), lambda i: (0, i, 0)),
            pl.BlockSpec((_HID, _HID), lambda i: (0, 0)),
            pl.BlockSpec((1, _HID), lambda i: (0, 0)),
            pl.BlockSpec((_HID, _HID), lambda i: (0, 0)),
            pl.BlockSpec((1, _HID), lambda i: (0, 0)),
        ],
        out_specs=pl.BlockSpec((2, _BR, _H2), lambda i: (0, i, 0)),
        out_shape=jax.ShapeDtypeStruct((2, _N, _H2), jnp.float32),
    )(z.reshape(2, _N, _H2), w1, b1.reshape(1, _HID), w2, b2.reshape(1, _HID))


def _mlp3_body(z_ref, w1_ref, b1_ref, w2_ref, b2_ref, bt_ref, l1w_ref,
               l1b_ref, l2w_ref, l2b_ref, out_ref, pooled):
    i = pl.program_id(0)

    @pl.when(i == 0)
    def _():
        pooled[...] = jnp.zeros((_G, _HID), jnp.float32)

    z0 = z_ref[0]
    z1 = z_ref[1]
    t = jnp.dot(z0, w1_ref[:_H2, :], preferred_element_type=jnp.float32)
    t = t + jnp.dot(z1, w1_ref[_H2:, :], preferred_element_type=jnp.float32)
    t = jnp.maximum(t + b1_ref[0][None, :], 0.0)
    y = jnp.dot(t, w2_ref[...], preferred_element_type=jnp.float32)
    y = jnp.maximum(y + b2_ref[0][None, :], 0.0)
    bt = bt_ref[0, 0, :]
    seg = lax.broadcasted_iota(jnp.int32, (_G, _BR), 0)
    onehot = (seg == bt[None, :]).astype(jnp.float32)
    pooled[...] += jnp.dot(onehot, y, preferred_element_type=jnp.float32)

    @pl.when(i == _N // _BR - 1)
    def _():
        p = pooled[...]
        u = jnp.dot(p, l1w_ref[...], preferred_element_type=jnp.float32)
        u = jnp.maximum(u + l1b_ref[0][None, :], 0.0)
        out_ref[...] = (
            jnp.dot(u, l2w_ref[...], preferred_element_type=jnp.float32)
            + l2b_ref[0][None, :]
        )


def _mlp3_pool_head(z, w1, b1, w2, b2, batch3, l1w, l1b, l2w_pad, l2b_pad):
    return pl.pallas_call(
        _mlp3_body,
        grid=(_N // _BR,),
        in_specs=[
            pl.BlockSpec((2, _BR, _H2), lambda i: (0, i, 0)),
            pl.BlockSpec((_HID, _HID), lambda i: (0, 0)),
            pl.BlockSpec((1, _HID), lambda i: (0, 0)),
            pl.BlockSpec((_HID, _HID), lambda i: (0, 0)),
            pl.BlockSpec((1, _HID), lambda i: (0, 0)),
            pl.BlockSpec((1, 1, _BR), lambda i: (i, 0, 0)),
            pl.BlockSpec((_HID, _HID), lambda i: (0, 0)),
            pl.BlockSpec((1, _HID), lambda i: (0, 0)),
            pl.BlockSpec((_HID, _H2), lambda i: (0, 0)),
            pl.BlockSpec((1, _H2), lambda i: (0, 0)),
        ],
        out_specs=pl.BlockSpec((_G, _H2), lambda i: (0, 0)),
        out_shape=jax.ShapeDtypeStruct((_G, _H2), jnp.float32),
        scratch_shapes=[pltpu.VMEM((_G, _HID), jnp.float32)],
    )(z.reshape(2, _N, _H2), w1, b1.reshape(1, _HID), w2, b2.reshape(1, _HID),
      batch3, l1w, l1b.reshape(1, _HID), l2w_pad, l2b_pad)


def kernel(x, edge_index, batch, c1_w1, c1_b1, c1_w2, c1_b2, c2_w1, c2_b1,
           c2_w2, c2_b2, c3_w1, c3_b1, c3_w2, c3_b2, l1_w, l1_b, l2_w, l2_b):
    src = edge_index[0]
    dst = edge_index[1]
    pad = _NCHUNK * _CH - _EPW
    src_p = jnp.pad(src.reshape(_NS, _EPW), ((0, 0), (0, pad)))
    dst_p = jnp.pad(dst.reshape(_NS, _EPW), ((0, 0), (0, pad)),
                    constant_values=_N)
    sidx = jnp.stack([src_p, src_p + _N]).reshape(2, _NS, _NCHUNK, _CH)
    didx = dst_p.reshape(_NS, _NCHUNK, _CH)
    batch3 = batch.reshape(_N // _BR, 1, _BR)
    l2w_pad = jnp.pad(l2_w, ((0, 0), (0, _H2 - _OUT)))
    l2b_pad = jnp.pad(l2_b, (0, _H2 - _OUT)).reshape(1, _H2)

    h = jnp.concatenate([x[:, :_H2], x[:, _H2:]], axis=0)
    z = _sc_agg(h, sidx, didx)
    h = _mlp(z, c1_w1, c1_b1, c1_w2, c1_b2).reshape(2 * _N, _H2)
    z = _sc_agg(h, sidx, didx)
    h = _mlp(z, c2_w1, c2_b1, c2_w2, c2_b2).reshape(2 * _N, _H2)
    z = _sc_agg(h, sidx, didx)
    out = _mlp3_pool_head(z, c3_w1, c3_b1, c3_w2, c3_b2, batch3,
                          l1_w, l1_b, l2w_pad, l2b_pad)
    return out[:, :_OUT]
